# baseline (speedup 1.0000x reference)
import functools

import jax
import jax.numpy as jnp
from jax import lax
from jax.experimental import pallas as pl
from jax.experimental.pallas import tpu as pltpu
from jax.experimental.pallas import tpu_sc as plsc

B, L, V, D, S = 128, 50, 100000, 100, 12
DP = 128
CA = 100
CW = 112
N_IN = B * L
NC, NS, LANES = 2, 16, 16
NW = NC * NS
PER = N_IN // NW
NB = PER * S
CH = 240
NCHUNK = NB // CH


def _sc1_body(inputs_ref, item_ref, tab_ref, ev0_out, item_out,
              idx_v, rows_v, sem):
    wid = lax.axis_index("s") * NC + lax.axis_index("c")
    base = wid * PER
    pltpu.sync_copy(inputs_ref.at[pl.ds(base, PER)], idx_v)
    pltpu.async_copy(tab_ref.at[idx_v], rows_v, sem).wait()
    pltpu.sync_copy(rows_v, ev0_out.at[pl.ds(base, PER)])
    pltpu.sync_copy(item_ref.at[pl.ds(base, PER)], idx_v)
    pltpu.async_copy(tab_ref.at[idx_v], rows_v, sem).wait()
    pltpu.sync_copy(rows_v, item_out.at[pl.ds(base, PER)])


@functools.cache
def _make_sc1():
    return functools.partial(
        pl.kernel,
        out_type=[
            jax.ShapeDtypeStruct((N_IN, DP), jnp.float32),
            jax.ShapeDtypeStruct((N_IN, DP), jnp.float32),
        ],
        mesh=plsc.VectorSubcoreMesh(core_axis_name="c", subcore_axis_name="s"),
        scratch_types=[
            pltpu.VMEM((PER,), jnp.int32),
            pltpu.VMEM((PER, DP), jnp.float32),
            pltpu.SemaphoreType.DMA,
        ],
        compiler_params=pltpu.CompilerParams(use_tc_tiling_on_sc=False),
    )(_sc1_body)


def _sc2_body(nflat_ref, tab_ref, ev1_out, idxc, buf, sem):
    wid = lax.axis_index("s") * NC + lax.axis_index("c")
    base = wid * NB
    for c in range(NCHUNK):
        pltpu.sync_copy(nflat_ref.at[pl.ds(base + c * CH, CH)], idxc)
        pltpu.async_copy(tab_ref.at[idxc], buf, sem).wait()
        pltpu.sync_copy(buf, ev1_out.at[pl.ds(base + c * CH, CH)])


@functools.cache
def _make_sc2():
    return functools.partial(
        pl.kernel,
        out_type=[
            jax.ShapeDtypeStruct((N_IN * S, DP), jnp.float32),
        ],
        mesh=plsc.VectorSubcoreMesh(core_axis_name="c", subcore_axis_name="s"),
        scratch_types=[
            pltpu.VMEM((CH,), jnp.int32),
            pltpu.VMEM((CH, DP), jnp.float32),
            pltpu.SemaphoreType.DMA,
        ],
        compiler_params=pltpu.CompilerParams(use_tc_tiling_on_sc=False),
    )(_sc2_body)


def _leaky(x):
    return jnp.where(x >= 0, x, 0.2 * x)


def _tc_body(ev0r, itemr, ev1r, adjr, maskr,
             atr, g1ar, g1br, g2r, g3ar, g3br, outr):
    f32 = jnp.float32
    ev0p = ev0r[0]
    ev0 = ev0p[:, :D]
    numv = ev0p[:, CW:CW + S]
    nrm = jnp.sqrt(jnp.sum(ev0 * ev0, axis=1, keepdims=True))
    h = ev0 / jnp.maximum(nrm, 1e-12)

    at = atr[...]
    hcat = jnp.concatenate(
        [h * at[0:1, :], h * at[1:2, :], h * at[2:3, :], h * at[3:4, :]], axis=0)
    E = lax.dot_general(hcat, h, (((1,), (1,)), ((), ())),
                        preferred_element_type=f32)
    E = _leaky(E)
    adjb = adjr[0]
    neg = jnp.full((L, L), -9e15, f32)
    al = jnp.where(adjb == 1, E[0:L], neg)
    al = jnp.where(adjb == 2, E[L:2 * L], al)
    al = jnp.where(adjb == 3, E[2 * L:3 * L], al)
    al = jnp.where(adjb == 4, E[3 * L:4 * L], al)
    mx = jnp.max(al, axis=1, keepdims=True)
    ex = jnp.exp(al - mx)
    al = ex / jnp.sum(ex, axis=1, keepdims=True)
    h_local = jnp.dot(al, h, preferred_element_type=f32)

    mf = maskr[0].astype(f32)
    sess = jnp.dot(mf, itemr[0][:, :D], preferred_element_type=f32) / jnp.sum(mf)

    g1a = g1ar[...]
    g1b = g1br[...]
    g2 = g2r[...]
    scores = []
    for s in range(S):
        evs = ev1r[0, :, s, :][:, :D]
        u = _leaky(jnp.dot(evs * sess, g1a, preferred_element_type=f32)
                   + numv[:, s:s + 1] * g1b)
        scores.append(jnp.dot(u, g2, preferred_element_type=f32))
    sc = jnp.concatenate(scores, axis=1)
    mx2 = jnp.max(sc, axis=1, keepdims=True)
    ex2 = jnp.exp(sc - mx2)
    al2 = ex2 / jnp.sum(ex2, axis=1, keepdims=True)
    nb = jnp.zeros((L, D), f32)
    for s in range(S):
        nb = nb + al2[:, s:s + 1] * ev1r[0, :, s, :][:, :D]
    outg = jnp.maximum(
        jnp.dot(ev0, g3ar[...], preferred_element_type=f32)
        + jnp.dot(nb, g3br[...], preferred_element_type=f32), 0.0)
    outr[0] = h_local + outg


def _tc_call(ev0, itemv, ev1, adj, maskv, at, g1a, g1b, g2, g3a, g3b):
    full = lambda shape: pl.BlockSpec(shape, lambda b: (0,) * len(shape))
    return pl.pallas_call(
        _tc_body,
        grid=(B,),
        in_specs=[
            pl.BlockSpec((1, L, DP), lambda b: (b, 0, 0)),
            pl.BlockSpec((1, L, DP), lambda b: (b, 0, 0)),
            pl.BlockSpec((1, L, S, DP), lambda b: (b, 0, 0, 0)),
            pl.BlockSpec((1, L, L), lambda b: (b, 0, 0)),
            pl.BlockSpec((1, 1, L), lambda b: (b, 0, 0)),
            full((4, D)),
            full((D, D)),
            full((1, D)),
            full((D, 1)),
            full((D, D)),
            full((D, D)),
        ],
        out_specs=pl.BlockSpec((1, L, D), lambda b: (b, 0, 0)),
        out_shape=jax.ShapeDtypeStruct((B, L, D), jnp.float32),
        compiler_params=pltpu.CompilerParams(
            dimension_semantics=("arbitrary",)),
    )(ev0, itemv, ev1, adj, maskv, at, g1a, g1b, g2, g3a, g3b)


def kernel(inputs, adj, mask_item, item, embedding, adj_all, num_w,
           a0, a1, a2, a3, g_w1, g_w2, g_w3):
    tab = jnp.concatenate(
        [embedding,
         lax.bitcast_convert_type(adj_all + jnp.int32(0x40000000), jnp.float32),
         num_w,
         jnp.zeros((V, DP - CW - S), jnp.float32)], axis=1)
    ev0w, itemw = _make_sc1()(inputs.reshape(-1), item.reshape(-1), tab)
    neighf = (lax.bitcast_convert_type(ev0w[:, CA:CA + S], jnp.int32)
              - jnp.int32(0x40000000))
    (ev1w,) = _make_sc2()(neighf.reshape(-1), tab)
    at = jnp.concatenate([a0, a1, a2, a3], axis=1).T
    return _tc_call(
        ev0w.reshape(B, L, DP),
        itemw.reshape(B, L, DP),
        ev1w.reshape(B, L, S, DP),
        adj,
        mask_item.reshape(B, 1, L),
        at,
        g_w1[:D], g_w1[D:], g_w2, g_w3[:D], g_w3[D:],
    )


# speedup vs baseline: 1.0618x; 1.0618x over previous
import functools

import jax
import jax.numpy as jnp
from jax import lax
from jax.experimental import pallas as pl
from jax.experimental.pallas import tpu as pltpu
from jax.experimental.pallas import tpu_sc as plsc

B, L, V, D, S = 128, 50, 100000, 100, 12
DP = 128
CA = 100
CW = 112
N_IN = B * L
NC, NS, LANES = 2, 16, 16
NW = NC * NS
PER = N_IN // NW
NB = PER * S
CH = 240
NCHUNK = NB // CH


def _sc1_body(inputs_ref, item_ref, tab_ref, ev0_out, item_out,
              idx_v, rows_v, sem):
    wid = lax.axis_index("s") * NC + lax.axis_index("c")
    base = wid * PER
    pltpu.sync_copy(inputs_ref.at[pl.ds(base, PER)], idx_v)
    pltpu.async_copy(tab_ref.at[idx_v], rows_v, sem).wait()
    pltpu.sync_copy(rows_v, ev0_out.at[pl.ds(base, PER)])
    pltpu.sync_copy(item_ref.at[pl.ds(base, PER)], idx_v)
    pltpu.async_copy(tab_ref.at[idx_v], rows_v, sem).wait()
    pltpu.sync_copy(rows_v, item_out.at[pl.ds(base, PER)])


@functools.cache
def _make_sc1():
    return functools.partial(
        pl.kernel,
        out_type=[
            jax.ShapeDtypeStruct((N_IN, DP), jnp.float32),
            jax.ShapeDtypeStruct((N_IN, DP), jnp.float32),
        ],
        mesh=plsc.VectorSubcoreMesh(core_axis_name="c", subcore_axis_name="s"),
        scratch_types=[
            pltpu.VMEM((PER,), jnp.int32),
            pltpu.VMEM((PER, DP), jnp.float32),
            pltpu.SemaphoreType.DMA,
        ],
        compiler_params=pltpu.CompilerParams(use_tc_tiling_on_sc=False),
    )(_sc1_body)


def _sc2_body(nflat_ref, tab_ref, ev1_out, idxc, buf, sem):
    wid = lax.axis_index("s") * NC + lax.axis_index("c")
    base = wid * NB
    for c in range(NCHUNK):
        pltpu.sync_copy(nflat_ref.at[pl.ds(base + c * CH, CH)], idxc)
        pltpu.async_copy(tab_ref.at[idxc], buf, sem).wait()
        pltpu.sync_copy(buf, ev1_out.at[pl.ds(base + c * CH, CH)])


@functools.cache
def _make_sc2():
    return functools.partial(
        pl.kernel,
        out_type=[
            jax.ShapeDtypeStruct((N_IN * S, DP), jnp.float32),
        ],
        mesh=plsc.VectorSubcoreMesh(core_axis_name="c", subcore_axis_name="s"),
        scratch_types=[
            pltpu.VMEM((CH,), jnp.int32),
            pltpu.VMEM((CH, DP), jnp.float32),
            pltpu.SemaphoreType.DMA,
        ],
        compiler_params=pltpu.CompilerParams(use_tc_tiling_on_sc=False),
    )(_sc2_body)


VB = 1000


def _pack_body(embr, adjr, numr, outr):
    outr[:, 0:D] = embr[...]
    outr[:, CA:CA + S] = lax.bitcast_convert_type(
        adjr[...] + jnp.int32(0x40000000), jnp.float32)
    outr[:, CW:CW + S] = numr[...]
    outr[:, CW + S:DP] = jnp.zeros((VB, DP - CW - S), jnp.float32)


def _pack_call(embedding, adj_all, num_w):
    return pl.pallas_call(
        _pack_body,
        grid=(V // VB,),
        in_specs=[
            pl.BlockSpec((VB, D), lambda i: (i, 0)),
            pl.BlockSpec((VB, S), lambda i: (i, 0)),
            pl.BlockSpec((VB, S), lambda i: (i, 0)),
        ],
        out_specs=pl.BlockSpec((VB, DP), lambda i: (i, 0)),
        out_shape=jax.ShapeDtypeStruct((V, DP), jnp.float32),
        compiler_params=pltpu.CompilerParams(
            dimension_semantics=("arbitrary",)),
    )(embedding, adj_all, num_w)


def _leaky(x):
    return jnp.where(x >= 0, x, 0.2 * x)


def _tc_body(ev0r, itemr, ev1r, adjr, maskr,
             atr, g1ar, g1br, g2r, g3ar, g3br, outr):
    f32 = jnp.float32
    ev0p = ev0r[0]
    ev0 = ev0p[:, :D]
    numv = ev0p[:, CW:CW + S]
    nrm = jnp.sqrt(jnp.sum(ev0 * ev0, axis=1, keepdims=True))
    h = ev0 / jnp.maximum(nrm, 1e-12)

    at = atr[...]
    hcat = jnp.concatenate(
        [h * at[0:1, :], h * at[1:2, :], h * at[2:3, :], h * at[3:4, :]], axis=0)
    E = lax.dot_general(hcat, h, (((1,), (1,)), ((), ())),
                        preferred_element_type=f32)
    E = _leaky(E)
    adjb = adjr[0]
    neg = jnp.full((L, L), -9e15, f32)
    al = jnp.where(adjb == 1, E[0:L], neg)
    al = jnp.where(adjb == 2, E[L:2 * L], al)
    al = jnp.where(adjb == 3, E[2 * L:3 * L], al)
    al = jnp.where(adjb == 4, E[3 * L:4 * L], al)
    mx = jnp.max(al, axis=1, keepdims=True)
    ex = jnp.exp(al - mx)
    al = ex / jnp.sum(ex, axis=1, keepdims=True)
    h_local = jnp.dot(al, h, preferred_element_type=f32)

    mf = maskr[0].astype(f32)
    sess = jnp.dot(mf, itemr[0][:, :D], preferred_element_type=f32) / jnp.sum(mf)

    g1a = g1ar[...]
    g1b = g1br[...]
    g2 = g2r[...]
    scores = []
    for s in range(S):
        evs = ev1r[0, :, s, :][:, :D]
        u = _leaky(jnp.dot(evs * sess, g1a, preferred_element_type=f32)
                   + numv[:, s:s + 1] * g1b)
        scores.append(jnp.dot(u, g2, preferred_element_type=f32))
    sc = jnp.concatenate(scores, axis=1)
    mx2 = jnp.max(sc, axis=1, keepdims=True)
    ex2 = jnp.exp(sc - mx2)
    al2 = ex2 / jnp.sum(ex2, axis=1, keepdims=True)
    nb = jnp.zeros((L, D), f32)
    for s in range(S):
        nb = nb + al2[:, s:s + 1] * ev1r[0, :, s, :][:, :D]
    outg = jnp.maximum(
        jnp.dot(ev0, g3ar[...], preferred_element_type=f32)
        + jnp.dot(nb, g3br[...], preferred_element_type=f32), 0.0)
    outr[0] = h_local + outg


def _tc_call(ev0, itemv, ev1, adj, maskv, at, g1a, g1b, g2, g3a, g3b):
    full = lambda shape: pl.BlockSpec(shape, lambda b: (0,) * len(shape))
    return pl.pallas_call(
        _tc_body,
        grid=(B,),
        in_specs=[
            pl.BlockSpec((1, L, DP), lambda b: (b, 0, 0)),
            pl.BlockSpec((1, L, DP), lambda b: (b, 0, 0)),
            pl.BlockSpec((1, L, S, DP), lambda b: (b, 0, 0, 0)),
            pl.BlockSpec((1, L, L), lambda b: (b, 0, 0)),
            pl.BlockSpec((1, 1, L), lambda b: (b, 0, 0)),
            full((4, D)),
            full((D, D)),
            full((1, D)),
            full((D, 1)),
            full((D, D)),
            full((D, D)),
        ],
        out_specs=pl.BlockSpec((1, L, D), lambda b: (b, 0, 0)),
        out_shape=jax.ShapeDtypeStruct((B, L, D), jnp.float32),
        compiler_params=pltpu.CompilerParams(
            dimension_semantics=("arbitrary",)),
    )(ev0, itemv, ev1, adj, maskv, at, g1a, g1b, g2, g3a, g3b)


def kernel(inputs, adj, mask_item, item, embedding, adj_all, num_w,
           a0, a1, a2, a3, g_w1, g_w2, g_w3):
    tab = _pack_call(embedding, adj_all, num_w)
    ev0w, itemw = _make_sc1()(inputs.reshape(-1), item.reshape(-1), tab)
    neighf = (lax.bitcast_convert_type(ev0w[:, CA:CA + S], jnp.int32)
              - jnp.int32(0x40000000))
    (ev1w,) = _make_sc2()(neighf.reshape(-1), tab)
    at = jnp.concatenate([a0, a1, a2, a3], axis=1).T
    return _tc_call(
        ev0w.reshape(B, L, DP),
        itemw.reshape(B, L, DP),
        ev1w.reshape(B, L, S, DP),
        adj,
        mask_item.reshape(B, 1, L),
        at,
        g_w1[:D], g_w1[D:], g_w2, g_w3[:D], g_w3[D:],
    )


# speedup vs baseline: 1.0875x; 1.0242x over previous
import functools

import jax
import jax.numpy as jnp
from jax import lax
from jax.experimental import pallas as pl
from jax.experimental.pallas import tpu as pltpu
from jax.experimental.pallas import tpu_sc as plsc

B, L, V, D, S = 128, 50, 100000, 100, 12
DP = 128
CA = 100
CW = 112
N_IN = B * L
NC, NS, LANES = 2, 16, 16
NW = NC * NS
PER = N_IN // NW
NB = PER * S
CH = 240
NCHUNK = NB // CH


def _sc1_body(inputs_ref, item_ref, tab_ref, ev0_out, item_out,
              idx_v, rows_v, sem):
    wid = lax.axis_index("s") * NC + lax.axis_index("c")
    base = wid * PER
    pltpu.sync_copy(inputs_ref.at[pl.ds(base, PER)], idx_v)
    pltpu.async_copy(tab_ref.at[idx_v], rows_v, sem).wait()
    pltpu.sync_copy(rows_v, ev0_out.at[pl.ds(base, PER)])
    pltpu.sync_copy(item_ref.at[pl.ds(base, PER)], idx_v)
    pltpu.async_copy(tab_ref.at[idx_v], rows_v, sem).wait()
    pltpu.sync_copy(rows_v, item_out.at[pl.ds(base, PER)])


@functools.cache
def _make_sc1():
    return functools.partial(
        pl.kernel,
        out_type=[
            jax.ShapeDtypeStruct((N_IN, DP), jnp.float32),
            jax.ShapeDtypeStruct((N_IN, DP), jnp.float32),
        ],
        mesh=plsc.VectorSubcoreMesh(core_axis_name="c", subcore_axis_name="s"),
        scratch_types=[
            pltpu.VMEM((PER,), jnp.int32),
            pltpu.VMEM((PER, DP), jnp.float32),
            pltpu.SemaphoreType.DMA,
        ],
        compiler_params=pltpu.CompilerParams(use_tc_tiling_on_sc=False),
    )(_sc1_body)


def _sc2_body(nflat_ref, tab_ref, ev1_out, idxc, buf, sem):
    wid = lax.axis_index("s") * NC + lax.axis_index("c")
    base = wid * NB
    for c in range(NCHUNK):
        pltpu.sync_copy(nflat_ref.at[pl.ds(base + c * CH, CH)], idxc)
        pltpu.async_copy(tab_ref.at[idxc], buf, sem).wait()
        pltpu.sync_copy(buf, ev1_out.at[pl.ds(base + c * CH, CH)])


@functools.cache
def _make_sc2():
    return functools.partial(
        pl.kernel,
        out_type=[
            jax.ShapeDtypeStruct((N_IN * S, DP), jnp.float32),
        ],
        mesh=plsc.VectorSubcoreMesh(core_axis_name="c", subcore_axis_name="s"),
        scratch_types=[
            pltpu.VMEM((CH,), jnp.int32),
            pltpu.VMEM((CH, DP), jnp.float32),
            pltpu.SemaphoreType.DMA,
        ],
        compiler_params=pltpu.CompilerParams(use_tc_tiling_on_sc=False),
    )(_sc2_body)


VB = 1000


def _pack_body(embr, adjr, numr, outr):
    outr[:, 0:D] = embr[...]
    outr[:, CA:CA + S] = lax.bitcast_convert_type(
        adjr[...] + jnp.int32(0x40000000), jnp.float32)
    outr[:, CW:CW + S] = numr[...]
    outr[:, CW + S:DP] = jnp.zeros((VB, DP - CW - S), jnp.float32)


def _pack_call(embedding, adj_all, num_w):
    return pl.pallas_call(
        _pack_body,
        grid=(V // VB,),
        in_specs=[
            pl.BlockSpec((VB, D), lambda i: (i, 0)),
            pl.BlockSpec((VB, S), lambda i: (i, 0)),
            pl.BlockSpec((VB, S), lambda i: (i, 0)),
        ],
        out_specs=pl.BlockSpec((VB, DP), lambda i: (i, 0)),
        out_shape=jax.ShapeDtypeStruct((V, DP), jnp.float32),
        compiler_params=pltpu.CompilerParams(
            dimension_semantics=("arbitrary",)),
    )(embedding, adj_all, num_w)


def _leaky(x):
    return jnp.where(x >= 0, x, 0.2 * x)


def _tc_body(ev0r, itemr, ev1r, adjr, maskr,
             atr, g1ar, g1br, g2r, g3ar, g3br, outr):
    f32 = jnp.float32
    ev0p = ev0r[0]
    ev0 = ev0p[:, :D]
    numv = ev0p[:, CW:CW + S]
    nrm = jnp.sqrt(jnp.sum(ev0 * ev0, axis=1, keepdims=True))
    h = ev0 / jnp.maximum(nrm, 1e-12)

    at = atr[...]
    hcat = jnp.concatenate(
        [h * at[0:1, :], h * at[1:2, :], h * at[2:3, :], h * at[3:4, :]], axis=0)
    E = lax.dot_general(hcat, h, (((1,), (1,)), ((), ())),
                        preferred_element_type=f32)
    E = _leaky(E)
    adjb = adjr[0]
    neg = jnp.full((L, L), -9e15, f32)
    al = jnp.where(adjb == 1, E[0:L], neg)
    al = jnp.where(adjb == 2, E[L:2 * L], al)
    al = jnp.where(adjb == 3, E[2 * L:3 * L], al)
    al = jnp.where(adjb == 4, E[3 * L:4 * L], al)
    mx = jnp.max(al, axis=1, keepdims=True)
    ex = jnp.exp(al - mx)
    al = ex / jnp.sum(ex, axis=1, keepdims=True)
    h_local = jnp.dot(al, h, preferred_element_type=f32)

    mf = maskr[0].astype(f32)
    sess = jnp.dot(mf, itemr[0][:, :D], preferred_element_type=f32) / jnp.sum(mf)

    g1a = g1ar[...]
    g1b = g1br[...]
    g2 = g2r[...]
    evcat = jnp.concatenate(
        [ev1r[0, :, s, :][:, :D] for s in range(S)], axis=0)
    wcol = jnp.concatenate(
        [numv[:, s:s + 1] for s in range(S)], axis=0)
    u = _leaky(jnp.dot(evcat * sess, g1a, preferred_element_type=f32)
               + wcol * g1b)
    scol = jnp.dot(u, g2, preferred_element_type=f32)
    sc = jnp.concatenate(
        [scol[s * L:(s + 1) * L] for s in range(S)], axis=1)
    mx2 = jnp.max(sc, axis=1, keepdims=True)
    ex2 = jnp.exp(sc - mx2)
    al2 = ex2 / jnp.sum(ex2, axis=1, keepdims=True)
    wcol2 = jnp.concatenate(
        [al2[:, s:s + 1] for s in range(S)], axis=0)
    weighted = evcat * wcol2
    nb = jnp.zeros((L, D), f32)
    for s in range(S):
        nb = nb + weighted[s * L:(s + 1) * L]
    outg = jnp.maximum(
        jnp.dot(ev0, g3ar[...], preferred_element_type=f32)
        + jnp.dot(nb, g3br[...], preferred_element_type=f32), 0.0)
    outr[0] = h_local + outg


def _tc_call(ev0, itemv, ev1, adj, maskv, at, g1a, g1b, g2, g3a, g3b):
    full = lambda shape: pl.BlockSpec(shape, lambda b: (0,) * len(shape))
    return pl.pallas_call(
        _tc_body,
        grid=(B,),
        in_specs=[
            pl.BlockSpec((1, L, DP), lambda b: (b, 0, 0)),
            pl.BlockSpec((1, L, DP), lambda b: (b, 0, 0)),
            pl.BlockSpec((1, L, S, DP), lambda b: (b, 0, 0, 0)),
            pl.BlockSpec((1, L, L), lambda b: (b, 0, 0)),
            pl.BlockSpec((1, 1, L), lambda b: (b, 0, 0)),
            full((4, D)),
            full((D, D)),
            full((1, D)),
            full((D, 1)),
            full((D, D)),
            full((D, D)),
        ],
        out_specs=pl.BlockSpec((1, L, D), lambda b: (b, 0, 0)),
        out_shape=jax.ShapeDtypeStruct((B, L, D), jnp.float32),
        compiler_params=pltpu.CompilerParams(
            dimension_semantics=("arbitrary",)),
    )(ev0, itemv, ev1, adj, maskv, at, g1a, g1b, g2, g3a, g3b)


def kernel(inputs, adj, mask_item, item, embedding, adj_all, num_w,
           a0, a1, a2, a3, g_w1, g_w2, g_w3):
    tab = _pack_call(embedding, adj_all, num_w)
    ev0w, itemw = _make_sc1()(inputs.reshape(-1), item.reshape(-1), tab)
    neighf = (lax.bitcast_convert_type(ev0w[:, CA:CA + S], jnp.int32)
              - jnp.int32(0x40000000))
    (ev1w,) = _make_sc2()(neighf.reshape(-1), tab)
    at = jnp.concatenate([a0, a1, a2, a3], axis=1).T
    return _tc_call(
        ev0w.reshape(B, L, DP),
        itemw.reshape(B, L, DP),
        ev1w.reshape(B, L, S, DP),
        adj,
        mask_item.reshape(B, 1, L),
        at,
        g_w1[:D], g_w1[D:], g_w2, g_w3[:D], g_w3[D:],
    )
